# baseline (speedup 1.0000x reference)
import functools

import jax
import jax.numpy as jnp
from jax import lax
from jax.experimental import pallas as pl
from jax.experimental.pallas import tpu as pltpu
from jax.experimental.pallas import tpu_sc as plsc

BATCH = 4
WINDOW = 8192
D_MODEL = 1024
NUM_CORES = 2
NUM_SUBCORES = 16
NUM_WORKERS = NUM_CORES * NUM_SUBCORES
ROWS_PER_WORKER = WINDOW // NUM_WORKERS
R = 2
STEPS = ROWS_PER_WORKER // R
NBUF = 12
K = 5
LANES = 16
CHUNKS = D_MODEL // LANES


def _body(x_hbm, t_hbm, out_hbm, buf, tbuf, in_sem, out_sem):
    wid = lax.axis_index("s") * NUM_CORES + lax.axis_index("c")
    base = wid * ROWS_PER_WORKER

    def in_copies(g, slot):
        w0 = base + g * R
        return [
            pltpu.make_async_copy(t_hbm.at[pl.ds(w0, R)], tbuf.at[slot],
                                  in_sem.at[slot]),
            pltpu.make_async_copy(x_hbm.at[:, pl.ds(w0, R)], buf.at[slot],
                                  in_sem.at[slot]),
        ]

    def out_copies(g, slot):
        w0 = base + g * R
        return [pltpu.make_async_copy(buf.at[slot],
                                      out_hbm.at[:, pl.ds(w0, R)],
                                      out_sem.at[slot])]

    def start(copies):
        for c in copies:
            c.start()

    def wait(copies):
        for c in copies:
            c.wait()

    UNROLL = 4

    def compute(slot):
        def chunk(c, carry):
            o0 = c * (UNROLL * LANES)
            for u in range(UNROLL):
                o = o0 + u * LANES
                for r in range(R):
                    t = tbuf[slot, r, pl.ds(o, LANES)]
                    for b in range(BATCH):
                        plsc.addupdate(buf.at[slot, b, r, pl.ds(o, LANES)], t)
            return carry

        lax.fori_loop(0, CHUNKS // UNROLL, chunk, 0)

    for p in range(K):
        start(in_copies(p, p))

    def superstep(it, carry):
        g0 = it * NBUF
        for j in range(NBUF):
            g = g0 + j
            pf_slot = (j + K) % NBUF

            @pl.when(g + K < STEPS)
            def _():
                @pl.when(g + K >= NBUF)
                def _():
                    wait(out_copies(g + K - NBUF, pf_slot))

                start(in_copies(g + K, pf_slot))

            wait(in_copies(g, j))
            compute(j)
            start(out_copies(g, j))
        return carry

    n_super = STEPS // NBUF
    lax.fori_loop(0, n_super, superstep, 0)

    for g in range(n_super * NBUF, STEPS):
        j = g % NBUF
        if g + K < STEPS:
            pf_slot = (j + K) % NBUF
            wait(out_copies(g + K - NBUF, pf_slot))
            start(in_copies(g + K, pf_slot))
        wait(in_copies(g, j))
        compute(j)
        start(out_copies(g, j))
    for gd in range(STEPS - NBUF, STEPS):
        wait(out_copies(gd, gd % NBUF))


@jax.jit
def kernel(x, pos_table):
    mesh = plsc.VectorSubcoreMesh(core_axis_name="c", subcore_axis_name="s")
    f = functools.partial(
        pl.kernel,
        mesh=mesh,
        out_type=jax.ShapeDtypeStruct((BATCH, WINDOW, D_MODEL), jnp.float32),
        scratch_types=[
            pltpu.VMEM((NBUF, BATCH, R, D_MODEL), jnp.float32),
            pltpu.VMEM((NBUF, R, D_MODEL), jnp.float32),
            pltpu.SemaphoreType.DMA((NBUF,)),
            pltpu.SemaphoreType.DMA((NBUF,)),
        ],
    )(_body)
    return f(x, pos_table)


# speedup vs baseline: 1.0374x; 1.0374x over previous
import functools

import jax
import jax.numpy as jnp
from jax import lax
from jax.experimental import pallas as pl
from jax.experimental.pallas import tpu as pltpu
from jax.experimental.pallas import tpu_sc as plsc

BATCH = 4
WINDOW = 8192
D_MODEL = 1024
NUM_CORES = 2
NUM_SUBCORES = 16
NUM_WORKERS = NUM_CORES * NUM_SUBCORES
ROWS_PER_WORKER = WINDOW // NUM_WORKERS
R = 2
STEPS = ROWS_PER_WORKER // R
NBUF = 8
K = 5
LANES = 16
CHUNKS = D_MODEL // LANES


def _body(x_hbm, t_hbm, out_hbm, buf, tbuf, in_sem, out_sem):
    wid = lax.axis_index("s") * NUM_CORES + lax.axis_index("c")
    base = wid * ROWS_PER_WORKER

    def in_copies(g, slot):
        w0 = base + g * R
        return [
            pltpu.make_async_copy(t_hbm.at[pl.ds(w0, R)], tbuf.at[slot],
                                  in_sem.at[slot]),
            pltpu.make_async_copy(x_hbm.at[:, pl.ds(w0, R)], buf.at[slot],
                                  in_sem.at[slot]),
        ]

    def out_copies(g, slot):
        w0 = base + g * R
        return [pltpu.make_async_copy(buf.at[slot],
                                      out_hbm.at[:, pl.ds(w0, R)],
                                      out_sem.at[slot])]

    def start(copies):
        for c in copies:
            c.start()

    def wait(copies):
        for c in copies:
            c.wait()

    UNROLL = 4

    def compute(slot):
        def chunk(c, carry):
            o0 = c * (UNROLL * LANES)
            for u in range(UNROLL):
                o = o0 + u * LANES
                for r in range(R):
                    t = tbuf[slot, r, pl.ds(o, LANES)]
                    for b in range(BATCH):
                        plsc.addupdate(buf.at[slot, b, r, pl.ds(o, LANES)], t)
            return carry

        lax.fori_loop(0, CHUNKS // UNROLL, chunk, 0)

    for p in range(K):
        start(in_copies(p, p))

    def superstep(it, carry):
        g0 = it * NBUF
        for j in range(NBUF):
            g = g0 + j
            pf_slot = (j + K) % NBUF

            @pl.when(g + K < STEPS)
            def _():
                @pl.when(g + K >= NBUF)
                def _():
                    wait(out_copies(g + K - NBUF, pf_slot))

                start(in_copies(g + K, pf_slot))

            wait(in_copies(g, j))
            compute(j)
            start(out_copies(g, j))
        return carry

    lax.fori_loop(0, STEPS // NBUF, superstep, 0)
    for gd in range(STEPS - NBUF, STEPS):
        wait(out_copies(gd, gd % NBUF))


@jax.jit
def kernel(x, pos_table):
    mesh = plsc.VectorSubcoreMesh(core_axis_name="c", subcore_axis_name="s")
    f = functools.partial(
        pl.kernel,
        mesh=mesh,
        out_type=jax.ShapeDtypeStruct((BATCH, WINDOW, D_MODEL), jnp.float32),
        scratch_types=[
            pltpu.VMEM((NBUF, BATCH, R, D_MODEL), jnp.float32),
            pltpu.VMEM((NBUF, R, D_MODEL), jnp.float32),
            pltpu.SemaphoreType.DMA((NBUF,)),
            pltpu.SemaphoreType.DMA((NBUF,)),
        ],
    )(_body)
    return f(x, pos_table)
